# Initial kernel scaffold; baseline (speedup 1.0000x reference)
import jax
import jax.numpy as jnp
from jax.experimental import pallas as pl


def kernel(atom_coords, atom_types, residue_indices, residue_types, atoms_per_residue, params):
    raise NotImplementedError("write your pallas kernel here")


# speedup vs baseline: 1.0626x; 1.0626x over previous
import jax
import jax.numpy as jnp
from jax.experimental import pallas as pl
from jax.experimental.pallas import tpu as pltpu

_CUTOFF = 0.3
_NUM_RBF = 16
_H = 128
_TI = 16
_TJ = 512
_PB = 1024
_NEG = -1e30


def _prep_kernel(at_ref, ridx_ref, rt_ref, aemb_ref, remb_ref, h0_ref):
    b = at_ref.shape[0]
    nt = aemb_ref.shape[0]
    oh_a = (at_ref[...] == jax.lax.broadcasted_iota(jnp.int32, (b, nt), 1))
    h0 = jnp.dot(oh_a.astype(jnp.float32), aemb_ref[...],
                 preferred_element_type=jnp.float32)
    nres = rt_ref.shape[0]
    nrt = remb_ref.shape[0]
    oh_t = (rt_ref[...] == jax.lax.broadcasted_iota(jnp.int32, (nres, nrt), 1))
    restab = jnp.dot(oh_t.astype(jnp.float32), remb_ref[...],
                     preferred_element_type=jnp.float32)
    oh_r = (ridx_ref[...] == jax.lax.broadcasted_iota(jnp.int32, (b, nres), 1))
    h0_ref[...] = h0 + jnp.dot(oh_r.astype(jnp.float32), restab,
                               preferred_element_type=jnp.float32)


def _ab_kernel(h_ref, w1a_ref, w1b_ref, be1_ref, ha_ref, hb_ref):
    h = h_ref[...]
    ha_ref[...] = (jnp.dot(h, w1a_ref[...], preferred_element_type=jnp.float32)
                   + be1_ref[...])
    hb_ref[...] = jnp.dot(h, w1b_ref[...], preferred_element_type=jnp.float32)


def _msg_kernel(x2i_ref, x2j_ref, ci_ref, cj_ref, ha_ref, hb_ref, wd_ref,
                wg_ref, we2_ref, be2_ref, agg_ref):
    ti, tj = hb_ref.shape[0], ha_ref.shape[0]
    ib, jb = pl.program_id(0), pl.program_id(1)
    f32 = jnp.float32
    ci = ci_ref[...].reshape(ti, 1, 3)
    cj = cj_ref[...].reshape(1, tj, 3)
    prod = jnp.dot(ci_ref[...], cj_ref[...].T, preferred_element_type=f32)
    d2g2 = (x2i_ref[...] + x2j_ref[...].reshape(1, tj) - 2.0 * prod)
    gi = ib * ti + jax.lax.broadcasted_iota(jnp.int32, (ti, 1, 1), 0)
    gj = jb * tj + jax.lax.broadcasted_iota(jnp.int32, (1, tj, 1), 1)
    mask = (d2g2.reshape(ti, tj, 1) < _CUTOFF * _CUTOFF) & (gi != gj)
    rel = ci - cj
    ed2 = jnp.sum(rel * rel, axis=2, keepdims=True)
    dist = jnp.sqrt(ed2 + 1e-12)
    centers = (jax.lax.broadcasted_iota(jnp.int32, (1, 1, _NUM_RBF), 2)
               .astype(f32) * (_CUTOFF / (_NUM_RBF - 1)))
    width = _CUTOFF / _NUM_RBF
    rbf = jnp.exp(-((dist - centers) ** 2) * (1.0 / (2.0 * width * width)))
    g = ed2 * wd_ref[...].reshape(1, 1, _H)
    g = g + jnp.dot(rbf.reshape(ti * tj, _NUM_RBF), wg_ref[...],
                    preferred_element_type=f32).reshape(ti, tj, _H)
    m1 = ha_ref[...].reshape(1, tj, _H) + hb_ref[...].reshape(ti, 1, _H) + g
    m1 = m1 * jax.nn.sigmoid(m1)
    m2 = (jnp.dot(m1.reshape(ti * tj, _H), we2_ref[...],
                  preferred_element_type=f32) + be2_ref[...])
    m2 = m2 * jax.nn.sigmoid(m2)
    m2 = jnp.where(mask, m2.reshape(ti, tj, _H), 0.0)
    acc = jnp.sum(m2, axis=1)

    @pl.when(jb == 0)
    def _():
        agg_ref[...] = jnp.zeros_like(agg_ref)

    agg_ref[...] += acc


def _node_kernel(h_ref, agg_ref, wn1a_ref, wn1b_ref, bn1_ref, wn2_ref,
                 bn2_ref, ho_ref):
    f32 = jnp.float32
    h = h_ref[...]
    t = (jnp.dot(h, wn1a_ref[...], preferred_element_type=f32)
         + jnp.dot(agg_ref[...], wn1b_ref[...], preferred_element_type=f32)
         + bn1_ref[...])
    t = t * jax.nn.sigmoid(t)
    u = jnp.dot(t, wn2_ref[...], preferred_element_type=f32) + bn2_ref[...]
    ho_ref[...] = h + u


def _pool1_kernel(h_ref, ridx_ref, rt_ref, remb_ref, wq_ref, bq_ref, wk_ref,
                  bk_ref, s_ref, smax_ref):
    f32 = jnp.float32
    b = h_ref.shape[0]
    nres = rt_ref.shape[0]
    nrt = remb_ref.shape[0]
    oh_t = (rt_ref[...] == jax.lax.broadcasted_iota(jnp.int32, (nres, nrt), 1))
    res_q = jnp.dot(oh_t.astype(f32), remb_ref[...], preferred_element_type=f32)
    queries = jnp.dot(res_q, wq_ref[...], preferred_element_type=f32) + bq_ref[...]
    oh_r = (ridx_ref[...] == jax.lax.broadcasted_iota(jnp.int32, (b, nres), 1))
    q_at = jnp.dot(oh_r.astype(f32), queries, preferred_element_type=f32)
    k = jnp.dot(h_ref[...], wk_ref[...], preferred_element_type=f32) + bk_ref[...]
    s = jnp.sum(q_at * k, axis=1, keepdims=True) * (_H ** -0.5)
    s_ref[...] = s
    oh_rt = (jax.lax.broadcasted_iota(jnp.int32, (nres, b), 0)
             == ridx_ref[...].reshape(1, b))
    masked = jnp.where(oh_rt, s.reshape(1, b), _NEG)
    bmax = jnp.max(masked, axis=1, keepdims=True)

    @pl.when(pl.program_id(0) == 0)
    def _():
        smax_ref[...] = jnp.full_like(smax_ref, _NEG)

    smax_ref[...] = jnp.maximum(smax_ref[...], bmax)


def _pool2_kernel(s_ref, smax_ref, h_ref, ridx_ref, wv_ref, bv_ref, pu_ref,
                  den_ref):
    f32 = jnp.float32
    b = h_ref.shape[0]
    nres = smax_ref.shape[0]
    oh_r = (ridx_ref[...] == jax.lax.broadcasted_iota(jnp.int32, (b, nres), 1))
    oh_rf = oh_r.astype(f32)
    smax_at = jnp.dot(oh_rf, smax_ref[...], preferred_element_type=f32)
    ex = jnp.exp(s_ref[...] - smax_at)
    v = jnp.dot(h_ref[...], wv_ref[...], preferred_element_type=f32) + bv_ref[...]
    oh_rt = oh_rf.T
    den_b = jnp.dot(oh_rt, ex, preferred_element_type=f32)
    pu_b = jnp.dot(oh_rt, ex * v, preferred_element_type=f32)

    @pl.when(pl.program_id(0) == 0)
    def _():
        pu_ref[...] = jnp.zeros_like(pu_ref)
        den_ref[...] = jnp.zeros_like(den_ref)

    pu_ref[...] += pu_b
    den_ref[...] += den_b


def _pool3_kernel(pu_ref, den_ref, wmu_ref, bmu_ref, wlv_ref, blv_ref,
                  mu_ref, lv_ref):
    f32 = jnp.float32
    den = den_ref[...]
    pooled = jnp.where(den > 0.0, pu_ref[...] / den, 0.0)
    mu_ref[...] = (jnp.dot(pooled, wmu_ref[...], preferred_element_type=f32)
                   + bmu_ref[...])
    lv = jnp.dot(pooled, wlv_ref[...], preferred_element_type=f32) + blv_ref[...]
    lv_ref[...] = jnp.clip(lv, -10.0, 2.0)


def kernel(atom_coords, atom_types, residue_indices, residue_types,
           atoms_per_residue, params):
    f32 = jnp.float32
    a = atom_coords.shape[0]
    nres = residue_types.shape[0]
    nrt = params["residue_embed"].shape[0]
    nt = params["atom_embed"].shape[0]
    lat = params["Wmu"].shape[1]
    at = atom_types.reshape(a, 1).astype(jnp.int32)
    ridx = residue_indices.reshape(a, 1).astype(jnp.int32)
    rt = residue_types.reshape(nres, 1).astype(jnp.int32)
    nb = a // _PB

    c = atom_coords - jnp.mean(atom_coords, axis=0, keepdims=True)
    x2 = jnp.sum(c * c, axis=-1, keepdims=True)

    h = pl.pallas_call(
        _prep_kernel,
        grid=(nb,),
        in_specs=[
            pl.BlockSpec((_PB, 1), lambda i: (i, 0)),
            pl.BlockSpec((_PB, 1), lambda i: (i, 0)),
            pl.BlockSpec((nres, 1), lambda i: (0, 0)),
            pl.BlockSpec((nt, _H), lambda i: (0, 0)),
            pl.BlockSpec((nrt, _H), lambda i: (0, 0)),
        ],
        out_specs=pl.BlockSpec((_PB, _H), lambda i: (i, 0)),
        out_shape=jax.ShapeDtypeStruct((a, _H), f32),
    )(at, ridx, rt, params["atom_embed"], params["residue_embed"])

    for lp in params["layers"]:
        w1a = lp["We1"][:_H]
        w1b = lp["We1"][_H:2 * _H]
        wd = lp["We1"][2 * _H:2 * _H + 1]
        wg = lp["We1"][2 * _H + 1:]
        be1 = lp["be1"].reshape(1, _H)
        be2 = lp["be2"].reshape(1, _H)

        ha, hb = pl.pallas_call(
            _ab_kernel,
            grid=(nb,),
            in_specs=[
                pl.BlockSpec((_PB, _H), lambda i: (i, 0)),
                pl.BlockSpec((_H, _H), lambda i: (0, 0)),
                pl.BlockSpec((_H, _H), lambda i: (0, 0)),
                pl.BlockSpec((1, _H), lambda i: (0, 0)),
            ],
            out_specs=[
                pl.BlockSpec((_PB, _H), lambda i: (i, 0)),
                pl.BlockSpec((_PB, _H), lambda i: (i, 0)),
            ],
            out_shape=[
                jax.ShapeDtypeStruct((a, _H), f32),
                jax.ShapeDtypeStruct((a, _H), f32),
            ],
        )(h, w1a, w1b, be1)

        agg = pl.pallas_call(
            _msg_kernel,
            grid=(a // _TI, a // _TJ),
            in_specs=[
                pl.BlockSpec((_TI, 1), lambda i, j: (i, 0)),
                pl.BlockSpec((_TJ, 1), lambda i, j: (j, 0)),
                pl.BlockSpec((_TI, 3), lambda i, j: (i, 0)),
                pl.BlockSpec((_TJ, 3), lambda i, j: (j, 0)),
                pl.BlockSpec((_TJ, _H), lambda i, j: (j, 0)),
                pl.BlockSpec((_TI, _H), lambda i, j: (i, 0)),
                pl.BlockSpec((1, _H), lambda i, j: (0, 0)),
                pl.BlockSpec((_NUM_RBF, _H), lambda i, j: (0, 0)),
                pl.BlockSpec((_H, _H), lambda i, j: (0, 0)),
                pl.BlockSpec((1, _H), lambda i, j: (0, 0)),
            ],
            out_specs=pl.BlockSpec((_TI, _H), lambda i, j: (i, 0)),
            out_shape=jax.ShapeDtypeStruct((a, _H), f32),
            compiler_params=pltpu.CompilerParams(
                dimension_semantics=("parallel", "arbitrary")),
        )(x2, x2, c, c, ha, hb, wd, wg, lp["We2"], be2)

        h = pl.pallas_call(
            _node_kernel,
            grid=(nb,),
            in_specs=[
                pl.BlockSpec((_PB, _H), lambda i: (i, 0)),
                pl.BlockSpec((_PB, _H), lambda i: (i, 0)),
                pl.BlockSpec((_H, _H), lambda i: (0, 0)),
                pl.BlockSpec((_H, _H), lambda i: (0, 0)),
                pl.BlockSpec((1, _H), lambda i: (0, 0)),
                pl.BlockSpec((_H, _H), lambda i: (0, 0)),
                pl.BlockSpec((1, _H), lambda i: (0, 0)),
            ],
            out_specs=pl.BlockSpec((_PB, _H), lambda i: (i, 0)),
            out_shape=jax.ShapeDtypeStruct((a, _H), f32),
        )(h, agg, lp["Wn1"][:_H], lp["Wn1"][_H:], lp["bn1"].reshape(1, _H),
          lp["Wn2"], lp["bn2"].reshape(1, _H))

    s, smax = pl.pallas_call(
        _pool1_kernel,
        grid=(nb,),
        in_specs=[
            pl.BlockSpec((_PB, _H), lambda i: (i, 0)),
            pl.BlockSpec((_PB, 1), lambda i: (i, 0)),
            pl.BlockSpec((nres, 1), lambda i: (0, 0)),
            pl.BlockSpec((nrt, _H), lambda i: (0, 0)),
            pl.BlockSpec((_H, _H), lambda i: (0, 0)),
            pl.BlockSpec((1, _H), lambda i: (0, 0)),
            pl.BlockSpec((_H, _H), lambda i: (0, 0)),
            pl.BlockSpec((1, _H), lambda i: (0, 0)),
        ],
        out_specs=[
            pl.BlockSpec((_PB, 1), lambda i: (i, 0)),
            pl.BlockSpec((nres, 1), lambda i: (0, 0)),
        ],
        out_shape=[
            jax.ShapeDtypeStruct((a, 1), f32),
            jax.ShapeDtypeStruct((nres, 1), f32),
        ],
    )(h, ridx, rt, params["residue_embed"], params["Wq"],
      params["bq"].reshape(1, _H), params["Wk"], params["bk"].reshape(1, _H))

    pu, den = pl.pallas_call(
        _pool2_kernel,
        grid=(nb,),
        in_specs=[
            pl.BlockSpec((_PB, 1), lambda i: (i, 0)),
            pl.BlockSpec((nres, 1), lambda i: (0, 0)),
            pl.BlockSpec((_PB, _H), lambda i: (i, 0)),
            pl.BlockSpec((_PB, 1), lambda i: (i, 0)),
            pl.BlockSpec((_H, _H), lambda i: (0, 0)),
            pl.BlockSpec((1, _H), lambda i: (0, 0)),
        ],
        out_specs=[
            pl.BlockSpec((nres, _H), lambda i: (0, 0)),
            pl.BlockSpec((nres, 1), lambda i: (0, 0)),
        ],
        out_shape=[
            jax.ShapeDtypeStruct((nres, _H), f32),
            jax.ShapeDtypeStruct((nres, 1), f32),
        ],
    )(s, smax, h, ridx, params["Wv"], params["bv"].reshape(1, _H))

    mu, lv = pl.pallas_call(
        _pool3_kernel,
        in_specs=[
            pl.BlockSpec((nres, _H), lambda: (0, 0)),
            pl.BlockSpec((nres, 1), lambda: (0, 0)),
            pl.BlockSpec((_H, lat), lambda: (0, 0)),
            pl.BlockSpec((1, lat), lambda: (0, 0)),
            pl.BlockSpec((_H, lat), lambda: (0, 0)),
            pl.BlockSpec((1, lat), lambda: (0, 0)),
        ],
        out_specs=[
            pl.BlockSpec((nres, lat), lambda: (0, 0)),
            pl.BlockSpec((nres, lat), lambda: (0, 0)),
        ],
        out_shape=[
            jax.ShapeDtypeStruct((nres, lat), f32),
            jax.ShapeDtypeStruct((nres, lat), f32),
        ],
    )(pu, den, params["Wmu"], params["bmu"].reshape(1, lat), params["Wlv"],
      params["blv"].reshape(1, lat))

    return mu, lv
